# baseline (speedup 1.0000x reference)
import functools

import jax
import jax.numpy as jnp
from jax import lax
from jax.experimental import pallas as pl
from jax.experimental.pallas import tpu as pltpu
from jax.experimental.pallas import tpu_sc as plsc

N = 10000
E = 320000
NCORES = 2
NSUB = 16
NW = NCORES * NSUB
EPW = E // NW
CHUNK = 80
NCHUNK = EPW // CHUNK
NPAD = 10240
ROWS_PER_TILE = NPAD // NSUB
D = 128
CK = 125
CR = 80


def _mesh():
    return plsc.VectorSubcoreMesh(
        core_axis_name="c", subcore_axis_name="s",
        num_cores=NCORES, num_subcores=NSUB)


NB = 5


def _gather_body(x_hbm, row3_hbm, out_hbm, idx2_v, r0, r1, r2, r3, r4,
                 sem_g, sem_w):
    c = lax.axis_index("c")
    s = lax.axis_index("s")
    wid = c * NSUB + s
    base0 = wid * EPW
    rows = [r0, r1, r2, r3, r4]
    pltpu.sync_copy(row3_hbm.at[wid], idx2_v)

    def outer(g, carry):
        j0 = g * NB
        for b in range(NB):
            pltpu.async_copy(x_hbm.at[idx2_v.at[j0 + b]], rows[b], sem_g)
        for b in range(NB):
            base = pl.multiple_of(base0 + (j0 + b) * CHUNK, CHUNK)
            pltpu.make_async_copy(
                x_hbm.at[idx2_v.at[j0 + b]], rows[b], sem_g).wait()
            pltpu.async_copy(rows[b], out_hbm.at[pl.ds(base, CHUNK)], sem_w)
        for b in range(NB):
            base = pl.multiple_of(base0 + (j0 + b) * CHUNK, CHUNK)
            pltpu.make_async_copy(
                rows[b], out_hbm.at[pl.ds(base, CHUNK)], sem_w).wait()
        return carry

    lax.fori_loop(0, NCHUNK // NB, outer, 0)


@functools.cache
def _gather_call():
    return pl.kernel(
        _gather_body,
        out_type=jax.ShapeDtypeStruct((E, D), jnp.float32),
        mesh=_mesh(),
        scratch_types=[
            pltpu.VMEM((NCHUNK, CHUNK), jnp.int32),
        ] + [pltpu.VMEM((CHUNK, D), jnp.float32)] * NB + [
            pltpu.SemaphoreType.DMA,
            pltpu.SemaphoreType.DMA,
        ],
    )


NBS = 3
NGRP = 41


def _scatter_body(h_hbm, col3_hbm, zeros_hbm, out_hbm, acc, idx2_v,
                  h0, h1, h2, sem_h, sem_s):
    c = lax.axis_index("c")
    s = lax.axis_index("s")
    wid = c * NSUB + s
    rbase = pl.multiple_of(s * ROWS_PER_TILE, ROWS_PER_TILE)
    hb = [h0, h1, h2]

    def zbody(k, carry):
        rb = pl.multiple_of(rbase + k * CHUNK, CHUNK)
        pltpu.sync_copy(zeros_hbm.at[pl.ds(rb, CHUNK)], h0)
        pltpu.sync_copy(h0, acc.at[pl.ds(rb, CHUNK)])
        return carry

    lax.fori_loop(0, ROWS_PER_TILE // CHUNK, zbody, 0)
    pltpu.sync_copy(col3_hbm.at[wid], idx2_v)
    plsc.subcore_barrier()

    base0 = wid * EPW

    def _load(j, b):
        base = pl.multiple_of(base0 + j * CHUNK, CHUNK)
        pltpu.async_copy(h_hbm.at[pl.ds(base, CHUNK)], hb[b], sem_h)

    def _wait_load(j, b):
        base = pl.multiple_of(base0 + j * CHUNK, CHUNK)
        pltpu.make_async_copy(h_hbm.at[pl.ds(base, CHUNK)], hb[b],
                              sem_h).wait()

    for b in range(NBS):
        _load(b, b)

    def outer(g, carry):
        j0 = g * NBS
        for b in range(NBS):
            _wait_load(j0 + b, b)
            pltpu.async_copy(hb[b], acc.at[idx2_v.at[j0 + b]], sem_s,
                             add=True)
        for b in range(NBS):
            pltpu.make_async_copy(
                hb[b], acc.at[idx2_v.at[j0 + b]], sem_s).wait()
            nxt = j0 + NBS + b

            @pl.when(nxt < NCHUNK)
            def _():
                _load(nxt, b)

        return carry

    lax.fori_loop(0, NGRP, outer, 0)
    for b in range(NCHUNK - NBS * NGRP):
        j = NGRP * NBS + b
        _wait_load(j, b)
        pltpu.async_copy(hb[b], acc.at[idx2_v.at[j]], sem_s, add=True)
    for b in range(NCHUNK - NBS * NGRP):
        j = NGRP * NBS + b
        pltpu.make_async_copy(hb[b], acc.at[idx2_v.at[j]], sem_s).wait()
    plsc.subcore_barrier()

    def wbody(k, carry):
        rb = pl.multiple_of(rbase + k * CHUNK, CHUNK)
        pltpu.sync_copy(acc.at[pl.ds(rb, CHUNK)], h0)
        pltpu.sync_copy(h0, out_hbm.at[c, pl.ds(rb, CHUNK)])
        return carry

    lax.fori_loop(0, ROWS_PER_TILE // CHUNK, wbody, 0)


@functools.cache
def _scatter_call():
    return pl.kernel(
        _scatter_body,
        out_type=jax.ShapeDtypeStruct((NCORES, NPAD, D), jnp.float32),
        mesh=_mesh(),
        scratch_types=[
            pltpu.VMEM_SHARED((NPAD, D), jnp.float32),
            pltpu.VMEM((NCHUNK, CHUNK), jnp.int32),
        ] + [pltpu.VMEM((CHUNK, D), jnp.float32)] * NBS + [
            pltpu.SemaphoreType.DMA,
            pltpu.SemaphoreType.DMA,
        ],
    )


BLK1 = 4000


def _mlp1_body(xg_ref, ea_ref, w1a_ref, b1a_ref, w1b_ref, b1b_ref, out_ref):
    w1a = w1a_ref[...].astype(jnp.bfloat16)
    m = jnp.dot(xg_ref[...].astype(jnp.bfloat16), w1a[:D],
                preferred_element_type=jnp.float32)
    m = m + jnp.dot(ea_ref[...].astype(jnp.bfloat16), w1a[D:],
                    preferred_element_type=jnp.float32)
    m = jnp.maximum(m + b1a_ref[...], 0.0).astype(jnp.bfloat16)
    h = jnp.dot(m, w1b_ref[...].astype(jnp.bfloat16),
                preferred_element_type=jnp.float32)
    out_ref[...] = h + b1b_ref[...]


def _mlp1_call(xg, ea, w1a, b1a, w1b, b1b, interpret=False):
    return pl.pallas_call(
        _mlp1_body,
        grid=(E // BLK1,),
        in_specs=[
            pl.BlockSpec((BLK1, D), lambda i: (i, 0)),
            pl.BlockSpec((BLK1, 16), lambda i: (i, 0)),
            pl.BlockSpec((D + 16, 256), lambda i: (0, 0)),
            pl.BlockSpec((1, 256), lambda i: (0, 0)),
            pl.BlockSpec((256, D), lambda i: (0, 0)),
            pl.BlockSpec((1, D), lambda i: (0, 0)),
        ],
        out_specs=pl.BlockSpec((BLK1, D), lambda i: (i, 0)),
        out_shape=jax.ShapeDtypeStruct((E, D), jnp.float32),
        interpret=interpret,
    )(xg, ea, w1a, b1a, w1b, b1b)


BLKC = 8000


def _counts_body(col_ref, cnt_ref):
    col = col_ref[...]
    hi = col // CK
    lo = col - hi * CK
    oh_hi = (hi == lax.broadcasted_iota(jnp.int32, (1, CR), 1)
             ).astype(jnp.bfloat16)
    oh_lo = (lo == lax.broadcasted_iota(jnp.int32, (1, D), 1)
             ).astype(jnp.bfloat16)
    c2 = lax.dot_general(oh_hi, oh_lo, (((0,), (0,)), ((), ())),
                         preferred_element_type=jnp.float32)

    @pl.when(pl.program_id(0) == 0)
    def _init():
        cnt_ref[...] = jnp.zeros((CR, D), jnp.float32)

    cnt_ref[...] = cnt_ref[...] + c2


def _counts_call(col2d, interpret=False):
    return pl.pallas_call(
        _counts_body,
        grid=(E // BLKC,),
        in_specs=[pl.BlockSpec((BLKC, 1), lambda i: (i, 0))],
        out_specs=pl.BlockSpec((CR, D), lambda i: (0, 0)),
        out_shape=jax.ShapeDtypeStruct((CR, D), jnp.float32),
        interpret=interpret,
    )(col2d)


BLK2 = 2000
CRB = BLK2 // CK


def _mlp2_body(p_ref, cnt_ref, x_ref, b_ref, u_ref, w2a_ref, b2a_ref,
               w2b_ref, b2b_ref, out_ref):
    sums = p_ref[0] + p_ref[1]
    j = lax.broadcasted_iota(jnp.int32, (BLK2, 1), 0)
    r = j // CK
    l = j - r * CK
    e1 = (r == lax.broadcasted_iota(jnp.int32, (1, CRB), 1)
          ).astype(jnp.float32)
    tmp = jnp.dot(e1, cnt_ref[...], preferred_element_type=jnp.float32)
    mask2 = l == lax.broadcasted_iota(jnp.int32, (1, D), 1)
    cnt = jnp.sum(jnp.where(mask2, tmp, 0.0), axis=1, keepdims=True)
    aggs = sums / jnp.maximum(cnt, 1.0)

    w2a = w2a_ref[...]
    m = jnp.dot(x_ref[...], w2a[:D], preferred_element_type=jnp.float32)
    m = m + jnp.dot(aggs, w2a[D:2 * D], preferred_element_type=jnp.float32)
    oh = (b_ref[...] == lax.broadcasted_iota(jnp.int32, (1, 16), 1)
          ).astype(jnp.float32)
    uw = jnp.dot(u_ref[...], w2a[2 * D:], preferred_element_type=jnp.float32)
    m = m + jnp.dot(oh, uw, preferred_element_type=jnp.float32)
    m = jnp.maximum(m + b2a_ref[...], 0.0)
    out = jnp.dot(m, w2b_ref[...], preferred_element_type=jnp.float32)
    out_ref[...] = out + b2b_ref[...]


def _mlp2_call(partials, cnt2d, x, batch2d, u, w2a, b2a, w2b, b2b,
               interpret=False):
    return pl.pallas_call(
        _mlp2_body,
        grid=(N // BLK2,),
        in_specs=[
            pl.BlockSpec((NCORES, BLK2, D), lambda i: (0, i, 0)),
            pl.BlockSpec((CRB, D), lambda i: (i, 0)),
            pl.BlockSpec((BLK2, D), lambda i: (i, 0)),
            pl.BlockSpec((BLK2, 1), lambda i: (i, 0)),
            pl.BlockSpec((16, 64), lambda i: (0, 0)),
            pl.BlockSpec((2 * D + 64, 256), lambda i: (0, 0)),
            pl.BlockSpec((1, 256), lambda i: (0, 0)),
            pl.BlockSpec((256, D), lambda i: (0, 0)),
            pl.BlockSpec((1, D), lambda i: (0, 0)),
        ],
        out_specs=pl.BlockSpec((BLK2, D), lambda i: (i, 0)),
        out_shape=jax.ShapeDtypeStruct((N, D), jnp.float32),
        interpret=interpret,
    )(partials, cnt2d, x, batch2d, u, w2a, b2a, w2b, b2b)


def kernel(x, edge_index, edge_attr, u, batch,
           W1a, b1a, W1b, b1b, W2a, b2a, W2b, b2b):
    row = edge_index[0]
    col = edge_index[1]
    row3 = row.reshape(NW, NCHUNK, CHUNK)
    col3 = col.reshape(NW, NCHUNK, CHUNK)
    cnt2d = _counts_call(col.reshape(-1, 1))
    xg = _gather_call()(x, row3)
    h = _mlp1_call(xg, edge_attr, W1a, b1a.reshape(1, -1),
                   W1b, b1b.reshape(1, -1))
    zeros_init = jnp.zeros((NPAD, D), jnp.float32)
    partials = _scatter_call()(h, col3, zeros_init)
    out = _mlp2_call(partials, cnt2d, x, batch.reshape(-1, 1), u,
                     W2a, b2a.reshape(1, -1), W2b, b2b.reshape(1, -1))
    return out


# speedup vs baseline: 1.0016x; 1.0016x over previous
import functools

import jax
import jax.numpy as jnp
from jax import lax
from jax.experimental import pallas as pl
from jax.experimental.pallas import tpu as pltpu
from jax.experimental.pallas import tpu_sc as plsc

N = 10000
E = 320000
NCORES = 2
NSUB = 16
NW = NCORES * NSUB
EPW = E // NW
CHUNK = 80
NCHUNK = EPW // CHUNK
NPAD = 10240
ROWS_PER_TILE = NPAD // NSUB
D = 128
CK = 125
CR = 80


def _mesh():
    return plsc.VectorSubcoreMesh(
        core_axis_name="c", subcore_axis_name="s",
        num_cores=NCORES, num_subcores=NSUB)


NB = 5


def _gather_body(x_hbm, row3_hbm, out_hbm, idx2_v, r0, r1, r2, r3, r4,
                 sem_g, sem_w):
    c = lax.axis_index("c")
    s = lax.axis_index("s")
    wid = c * NSUB + s
    base0 = wid * EPW
    rows = [r0, r1, r2, r3, r4]
    pltpu.sync_copy(row3_hbm.at[wid], idx2_v)

    def _wb(j, b):
        base = pl.multiple_of(base0 + j * CHUNK, CHUNK)
        return pltpu.make_async_copy(
            rows[b], out_hbm.at[pl.ds(base, CHUNK)], sem_w)

    for b in range(NB):
        pltpu.async_copy(x_hbm.at[idx2_v.at[b]], rows[b], sem_g)

    def outer(g, carry):
        j0 = g * NB
        for b in range(NB):
            j = j0 + b
            base = pl.multiple_of(base0 + j * CHUNK, CHUNK)
            pltpu.make_async_copy(
                x_hbm.at[idx2_v.at[j]], rows[b], sem_g).wait()
            pltpu.async_copy(rows[b], out_hbm.at[pl.ds(base, CHUNK)], sem_w)
        for b in range(NB):
            j = j0 + b
            _wb(j, b).wait()
            nxt = j + NB

            @pl.when(nxt < NCHUNK)
            def _():
                pltpu.async_copy(x_hbm.at[idx2_v.at[nxt]], rows[b], sem_g)

        return carry

    lax.fori_loop(0, NCHUNK // NB, outer, 0)


@functools.cache
def _gather_call():
    return pl.kernel(
        _gather_body,
        out_type=jax.ShapeDtypeStruct((E, D), jnp.float32),
        mesh=_mesh(),
        scratch_types=[
            pltpu.VMEM((NCHUNK, CHUNK), jnp.int32),
        ] + [pltpu.VMEM((CHUNK, D), jnp.float32)] * NB + [
            pltpu.SemaphoreType.DMA,
            pltpu.SemaphoreType.DMA,
        ],
    )


NBS = 3
NGRP = 41


def _scatter_body(h_hbm, col3_hbm, zeros_hbm, out_hbm, acc, idx2_v,
                  h0, h1, h2, sem_h, sem_s):
    c = lax.axis_index("c")
    s = lax.axis_index("s")
    wid = c * NSUB + s
    rbase = pl.multiple_of(s * ROWS_PER_TILE, ROWS_PER_TILE)
    hb = [h0, h1, h2]

    def zbody(k, carry):
        rb = pl.multiple_of(rbase + k * CHUNK, CHUNK)
        pltpu.sync_copy(zeros_hbm.at[pl.ds(rb, CHUNK)], h0)
        pltpu.sync_copy(h0, acc.at[pl.ds(rb, CHUNK)])
        return carry

    lax.fori_loop(0, ROWS_PER_TILE // CHUNK, zbody, 0)
    pltpu.sync_copy(col3_hbm.at[wid], idx2_v)
    plsc.subcore_barrier()

    base0 = wid * EPW

    def _load(j, b):
        base = pl.multiple_of(base0 + j * CHUNK, CHUNK)
        pltpu.async_copy(h_hbm.at[pl.ds(base, CHUNK)], hb[b], sem_h)

    def _wait_load(j, b):
        base = pl.multiple_of(base0 + j * CHUNK, CHUNK)
        pltpu.make_async_copy(h_hbm.at[pl.ds(base, CHUNK)], hb[b],
                              sem_h).wait()

    for b in range(NBS):
        _load(b, b)

    def outer(g, carry):
        j0 = g * NBS
        for b in range(NBS):
            _wait_load(j0 + b, b)
            pltpu.async_copy(hb[b], acc.at[idx2_v.at[j0 + b]], sem_s,
                             add=True)
        for b in range(NBS):
            pltpu.make_async_copy(
                hb[b], acc.at[idx2_v.at[j0 + b]], sem_s).wait()
            nxt = j0 + NBS + b

            @pl.when(nxt < NCHUNK)
            def _():
                _load(nxt, b)

        return carry

    lax.fori_loop(0, NGRP, outer, 0)
    for b in range(NCHUNK - NBS * NGRP):
        j = NGRP * NBS + b
        _wait_load(j, b)
        pltpu.async_copy(hb[b], acc.at[idx2_v.at[j]], sem_s, add=True)
    for b in range(NCHUNK - NBS * NGRP):
        j = NGRP * NBS + b
        pltpu.make_async_copy(hb[b], acc.at[idx2_v.at[j]], sem_s).wait()
    plsc.subcore_barrier()

    def wbody(k, carry):
        rb = pl.multiple_of(rbase + k * CHUNK, CHUNK)
        pltpu.sync_copy(acc.at[pl.ds(rb, CHUNK)], h0)
        pltpu.sync_copy(h0, out_hbm.at[c, pl.ds(rb, CHUNK)])
        return carry

    lax.fori_loop(0, ROWS_PER_TILE // CHUNK, wbody, 0)


@functools.cache
def _scatter_call():
    return pl.kernel(
        _scatter_body,
        out_type=jax.ShapeDtypeStruct((NCORES, NPAD, D), jnp.float32),
        mesh=_mesh(),
        scratch_types=[
            pltpu.VMEM_SHARED((NPAD, D), jnp.float32),
            pltpu.VMEM((NCHUNK, CHUNK), jnp.int32),
        ] + [pltpu.VMEM((CHUNK, D), jnp.float32)] * NBS + [
            pltpu.SemaphoreType.DMA,
            pltpu.SemaphoreType.DMA,
        ],
    )


BLK1 = 4000


def _mlp1_body(xg_ref, ea_ref, w1a_ref, b1a_ref, w1b_ref, b1b_ref, out_ref):
    w1a = w1a_ref[...].astype(jnp.bfloat16)
    m = jnp.dot(xg_ref[...].astype(jnp.bfloat16), w1a[:D],
                preferred_element_type=jnp.float32)
    m = m + jnp.dot(ea_ref[...].astype(jnp.bfloat16), w1a[D:],
                    preferred_element_type=jnp.float32)
    m = jnp.maximum(m + b1a_ref[...], 0.0).astype(jnp.bfloat16)
    h = jnp.dot(m, w1b_ref[...].astype(jnp.bfloat16),
                preferred_element_type=jnp.float32)
    out_ref[...] = h + b1b_ref[...]


def _mlp1_call(xg, ea, w1a, b1a, w1b, b1b, interpret=False):
    return pl.pallas_call(
        _mlp1_body,
        grid=(E // BLK1,),
        in_specs=[
            pl.BlockSpec((BLK1, D), lambda i: (i, 0)),
            pl.BlockSpec((BLK1, 16), lambda i: (i, 0)),
            pl.BlockSpec((D + 16, 256), lambda i: (0, 0)),
            pl.BlockSpec((1, 256), lambda i: (0, 0)),
            pl.BlockSpec((256, D), lambda i: (0, 0)),
            pl.BlockSpec((1, D), lambda i: (0, 0)),
        ],
        out_specs=pl.BlockSpec((BLK1, D), lambda i: (i, 0)),
        out_shape=jax.ShapeDtypeStruct((E, D), jnp.float32),
        interpret=interpret,
    )(xg, ea, w1a, b1a, w1b, b1b)


BLKC = 8000


def _counts_body(col_ref, cnt_ref):
    col = col_ref[...]
    hi = col // CK
    lo = col - hi * CK
    oh_hi = (hi == lax.broadcasted_iota(jnp.int32, (1, CR), 1)
             ).astype(jnp.bfloat16)
    oh_lo = (lo == lax.broadcasted_iota(jnp.int32, (1, D), 1)
             ).astype(jnp.bfloat16)
    c2 = lax.dot_general(oh_hi, oh_lo, (((0,), (0,)), ((), ())),
                         preferred_element_type=jnp.float32)

    @pl.when(pl.program_id(0) == 0)
    def _init():
        cnt_ref[...] = jnp.zeros((CR, D), jnp.float32)

    cnt_ref[...] = cnt_ref[...] + c2


def _counts_call(col2d, interpret=False):
    return pl.pallas_call(
        _counts_body,
        grid=(E // BLKC,),
        in_specs=[pl.BlockSpec((BLKC, 1), lambda i: (i, 0))],
        out_specs=pl.BlockSpec((CR, D), lambda i: (0, 0)),
        out_shape=jax.ShapeDtypeStruct((CR, D), jnp.float32),
        interpret=interpret,
    )(col2d)


BLK2 = 2000
CRB = BLK2 // CK


def _mlp2_body(p_ref, cnt_ref, x_ref, b_ref, u_ref, w2a_ref, b2a_ref,
               w2b_ref, b2b_ref, out_ref):
    sums = p_ref[0] + p_ref[1]
    j = lax.broadcasted_iota(jnp.int32, (BLK2, 1), 0)
    r = j // CK
    l = j - r * CK
    e1 = (r == lax.broadcasted_iota(jnp.int32, (1, CRB), 1)
          ).astype(jnp.float32)
    tmp = jnp.dot(e1, cnt_ref[...], preferred_element_type=jnp.float32)
    mask2 = l == lax.broadcasted_iota(jnp.int32, (1, D), 1)
    cnt = jnp.sum(jnp.where(mask2, tmp, 0.0), axis=1, keepdims=True)
    aggs = sums / jnp.maximum(cnt, 1.0)

    w2a = w2a_ref[...]
    m = jnp.dot(x_ref[...], w2a[:D], preferred_element_type=jnp.float32)
    m = m + jnp.dot(aggs, w2a[D:2 * D], preferred_element_type=jnp.float32)
    oh = (b_ref[...] == lax.broadcasted_iota(jnp.int32, (1, 16), 1)
          ).astype(jnp.float32)
    uw = jnp.dot(u_ref[...], w2a[2 * D:], preferred_element_type=jnp.float32)
    m = m + jnp.dot(oh, uw, preferred_element_type=jnp.float32)
    m = jnp.maximum(m + b2a_ref[...], 0.0)
    out = jnp.dot(m, w2b_ref[...], preferred_element_type=jnp.float32)
    out_ref[...] = out + b2b_ref[...]


def _mlp2_call(partials, cnt2d, x, batch2d, u, w2a, b2a, w2b, b2b,
               interpret=False):
    return pl.pallas_call(
        _mlp2_body,
        grid=(N // BLK2,),
        in_specs=[
            pl.BlockSpec((NCORES, BLK2, D), lambda i: (0, i, 0)),
            pl.BlockSpec((CRB, D), lambda i: (i, 0)),
            pl.BlockSpec((BLK2, D), lambda i: (i, 0)),
            pl.BlockSpec((BLK2, 1), lambda i: (i, 0)),
            pl.BlockSpec((16, 64), lambda i: (0, 0)),
            pl.BlockSpec((2 * D + 64, 256), lambda i: (0, 0)),
            pl.BlockSpec((1, 256), lambda i: (0, 0)),
            pl.BlockSpec((256, D), lambda i: (0, 0)),
            pl.BlockSpec((1, D), lambda i: (0, 0)),
        ],
        out_specs=pl.BlockSpec((BLK2, D), lambda i: (i, 0)),
        out_shape=jax.ShapeDtypeStruct((N, D), jnp.float32),
        interpret=interpret,
    )(partials, cnt2d, x, batch2d, u, w2a, b2a, w2b, b2b)


def kernel(x, edge_index, edge_attr, u, batch,
           W1a, b1a, W1b, b1b, W2a, b2a, W2b, b2b):
    row = edge_index[0]
    col = edge_index[1]
    row3 = row.reshape(NW, NCHUNK, CHUNK)
    col3 = col.reshape(NW, NCHUNK, CHUNK)
    xg = _gather_call()(x, row3)
    cnt2d = _counts_call(col.reshape(-1, 1))
    h = _mlp1_call(xg, edge_attr, W1a, b1a.reshape(1, -1),
                   W1b, b1b.reshape(1, -1))
    zeros_init = jnp.zeros((NPAD, D), jnp.float32)
    partials = _scatter_call()(h, col3, zeros_init)
    out = _mlp2_call(partials, cnt2d, x, batch.reshape(-1, 1), u,
                     W2a, b2a.reshape(1, -1), W2b, b2b.reshape(1, -1))
    return out


# speedup vs baseline: 1.0376x; 1.0360x over previous
import functools

import jax
import jax.numpy as jnp
from jax import lax
from jax.experimental import pallas as pl
from jax.experimental.pallas import tpu as pltpu
from jax.experimental.pallas import tpu_sc as plsc

N = 10000
E = 320000
NCORES = 2
NSUB = 16
NW = NCORES * NSUB
EPW = E // NW
CHUNK = 80
NCHUNK = EPW // CHUNK
NPAD = 10240
ROWS_PER_TILE = NPAD // NSUB
D = 128
CK = 125
CR = 80


def _mesh():
    return plsc.VectorSubcoreMesh(
        core_axis_name="c", subcore_axis_name="s",
        num_cores=NCORES, num_subcores=NSUB)


NB = 5


def _gather_body(x_hbm, row3_hbm, out_hbm, idx2_v, r0, r1, r2, r3, r4,
                 sem_g, sem_w):
    c = lax.axis_index("c")
    s = lax.axis_index("s")
    wid = c * NSUB + s
    base0 = wid * EPW
    rows = [r0, r1, r2, r3, r4]
    pltpu.sync_copy(row3_hbm.at[wid], idx2_v)

    def _wb(j, b):
        base = pl.multiple_of(base0 + j * CHUNK, CHUNK)
        return pltpu.make_async_copy(
            rows[b], out_hbm.at[pl.ds(base, CHUNK)], sem_w)

    for b in range(NB):
        pltpu.async_copy(x_hbm.at[idx2_v.at[b]], rows[b], sem_g)

    def outer(g, carry):
        j0 = g * NB
        for b in range(NB):
            j = j0 + b
            base = pl.multiple_of(base0 + j * CHUNK, CHUNK)
            pltpu.make_async_copy(
                x_hbm.at[idx2_v.at[j]], rows[b], sem_g).wait()
            pltpu.async_copy(rows[b], out_hbm.at[pl.ds(base, CHUNK)], sem_w)
        for b in range(NB):
            j = j0 + b
            _wb(j, b).wait()
            nxt = j + NB

            @pl.when(nxt < NCHUNK)
            def _():
                pltpu.async_copy(x_hbm.at[idx2_v.at[nxt]], rows[b], sem_g)

        return carry

    lax.fori_loop(0, NCHUNK // NB, outer, 0)


@functools.cache
def _gather_call():
    return pl.kernel(
        _gather_body,
        out_type=jax.ShapeDtypeStruct((E, D), jnp.float32),
        mesh=_mesh(),
        scratch_types=[
            pltpu.VMEM((NCHUNK, CHUNK), jnp.int32),
        ] + [pltpu.VMEM((CHUNK, D), jnp.float32)] * NB + [
            pltpu.SemaphoreType.DMA,
            pltpu.SemaphoreType.DMA,
        ],
    )


NBS = 3
NGRP = 41


def _scatter_body(h_hbm, col3_hbm, zeros_hbm, out_hbm, acc, idx2_v,
                  h0, h1, h2, sem_h, sem_s):
    c = lax.axis_index("c")
    s = lax.axis_index("s")
    wid = c * NSUB + s
    rbase = pl.multiple_of(s * ROWS_PER_TILE, ROWS_PER_TILE)
    hb = [h0, h1, h2]

    def zbody(k, carry):
        rb = pl.multiple_of(rbase + k * CHUNK, CHUNK)
        pltpu.sync_copy(zeros_hbm.at[pl.ds(rb, CHUNK)], h0)
        pltpu.sync_copy(h0, acc.at[pl.ds(rb, CHUNK)])
        return carry

    lax.fori_loop(0, ROWS_PER_TILE // CHUNK, zbody, 0)
    pltpu.sync_copy(col3_hbm.at[wid], idx2_v)
    plsc.subcore_barrier()

    base0 = wid * EPW

    def _load(j, b):
        base = pl.multiple_of(base0 + j * CHUNK, CHUNK)
        pltpu.async_copy(h_hbm.at[pl.ds(base, CHUNK)], hb[b], sem_h)

    def _wait_load(j, b):
        base = pl.multiple_of(base0 + j * CHUNK, CHUNK)
        pltpu.make_async_copy(h_hbm.at[pl.ds(base, CHUNK)], hb[b],
                              sem_h).wait()

    for b in range(NBS):
        _load(b, b)

    def outer(g, carry):
        j0 = g * NBS
        for b in range(NBS):
            _wait_load(j0 + b, b)
            pltpu.async_copy(hb[b], acc.at[idx2_v.at[j0 + b]], sem_s,
                             add=True)
        for b in range(NBS):
            pltpu.make_async_copy(
                hb[b], acc.at[idx2_v.at[j0 + b]], sem_s).wait()
            nxt = j0 + NBS + b

            @pl.when(nxt < NCHUNK)
            def _():
                _load(nxt, b)

        return carry

    lax.fori_loop(0, NGRP, outer, 0)
    for b in range(NCHUNK - NBS * NGRP):
        j = NGRP * NBS + b
        _wait_load(j, b)
        pltpu.async_copy(hb[b], acc.at[idx2_v.at[j]], sem_s, add=True)
    for b in range(NCHUNK - NBS * NGRP):
        j = NGRP * NBS + b
        pltpu.make_async_copy(hb[b], acc.at[idx2_v.at[j]], sem_s).wait()
    plsc.subcore_barrier()

    def wbody(k, carry):
        rb = pl.multiple_of(rbase + k * CHUNK, CHUNK)
        pltpu.sync_copy(acc.at[pl.ds(rb, CHUNK)], h0)
        pltpu.sync_copy(h0, out_hbm.at[c, pl.ds(rb, CHUNK)])
        return carry

    lax.fori_loop(0, ROWS_PER_TILE // CHUNK, wbody, 0)


@functools.cache
def _scatter_call():
    return pl.kernel(
        _scatter_body,
        out_type=jax.ShapeDtypeStruct((NCORES, NPAD, D), jnp.float32),
        mesh=_mesh(),
        scratch_types=[
            pltpu.VMEM_SHARED((NPAD, D), jnp.float32),
            pltpu.VMEM((NCHUNK, CHUNK), jnp.int32),
        ] + [pltpu.VMEM((CHUNK, D), jnp.float32)] * NBS + [
            pltpu.SemaphoreType.DMA,
            pltpu.SemaphoreType.DMA,
        ],
    )


BLK1 = 4000


def _mlp1_body(xg_ref, ea_ref, col_ref, w1a_ref, b1a_ref, w1b_ref, b1b_ref,
               out_ref, cnt_ref):
    w1a = w1a_ref[...].astype(jnp.bfloat16)
    m = jnp.dot(xg_ref[...].astype(jnp.bfloat16), w1a[:D],
                preferred_element_type=jnp.float32)
    m = m + jnp.dot(ea_ref[...].astype(jnp.bfloat16), w1a[D:],
                    preferred_element_type=jnp.float32)
    m = jnp.maximum(m + b1a_ref[...], 0.0).astype(jnp.bfloat16)
    h = jnp.dot(m, w1b_ref[...].astype(jnp.bfloat16),
                preferred_element_type=jnp.float32)
    out_ref[...] = h + b1b_ref[...]

    col = col_ref[...]
    hi = col // CK
    lo = col - hi * CK
    oh_hi = (hi == lax.broadcasted_iota(jnp.int32, (1, CR), 1)
             ).astype(jnp.bfloat16)
    oh_lo = (lo == lax.broadcasted_iota(jnp.int32, (1, D), 1)
             ).astype(jnp.bfloat16)
    c2 = lax.dot_general(oh_hi, oh_lo, (((0,), (0,)), ((), ())),
                         preferred_element_type=jnp.float32)

    @pl.when(pl.program_id(0) == 0)
    def _init():
        cnt_ref[...] = jnp.zeros((CR, D), jnp.float32)

    cnt_ref[...] = cnt_ref[...] + c2


def _mlp1_call(xg, ea, col2d, w1a, b1a, w1b, b1b, interpret=False):
    return pl.pallas_call(
        _mlp1_body,
        grid=(E // BLK1,),
        in_specs=[
            pl.BlockSpec((BLK1, D), lambda i: (i, 0)),
            pl.BlockSpec((BLK1, 16), lambda i: (i, 0)),
            pl.BlockSpec((BLK1, 1), lambda i: (i, 0)),
            pl.BlockSpec((D + 16, 256), lambda i: (0, 0)),
            pl.BlockSpec((1, 256), lambda i: (0, 0)),
            pl.BlockSpec((256, D), lambda i: (0, 0)),
            pl.BlockSpec((1, D), lambda i: (0, 0)),
        ],
        out_specs=[
            pl.BlockSpec((BLK1, D), lambda i: (i, 0)),
            pl.BlockSpec((CR, D), lambda i: (0, 0)),
        ],
        out_shape=[
            jax.ShapeDtypeStruct((E, D), jnp.float32),
            jax.ShapeDtypeStruct((CR, D), jnp.float32),
        ],
        interpret=interpret,
    )(xg, ea, col2d, w1a, b1a, w1b, b1b)


BLK2 = 2000
CRB = BLK2 // CK


def _mlp2_body(p_ref, cnt_ref, x_ref, b_ref, u_ref, w2a_ref, b2a_ref,
               w2b_ref, b2b_ref, out_ref):
    sums = p_ref[0] + p_ref[1]
    j = lax.broadcasted_iota(jnp.int32, (BLK2, 1), 0)
    r = j // CK
    l = j - r * CK
    e1 = (r == lax.broadcasted_iota(jnp.int32, (1, CRB), 1)
          ).astype(jnp.float32)
    tmp = jnp.dot(e1, cnt_ref[...], preferred_element_type=jnp.float32)
    mask2 = l == lax.broadcasted_iota(jnp.int32, (1, D), 1)
    cnt = jnp.sum(jnp.where(mask2, tmp, 0.0), axis=1, keepdims=True)
    aggs = sums / jnp.maximum(cnt, 1.0)

    w2a = w2a_ref[...]
    m = jnp.dot(x_ref[...], w2a[:D], preferred_element_type=jnp.float32)
    m = m + jnp.dot(aggs, w2a[D:2 * D], preferred_element_type=jnp.float32)
    oh = (b_ref[...] == lax.broadcasted_iota(jnp.int32, (1, 16), 1)
          ).astype(jnp.float32)
    uw = jnp.dot(u_ref[...], w2a[2 * D:], preferred_element_type=jnp.float32)
    m = m + jnp.dot(oh, uw, preferred_element_type=jnp.float32)
    m = jnp.maximum(m + b2a_ref[...], 0.0)
    out = jnp.dot(m, w2b_ref[...], preferred_element_type=jnp.float32)
    out_ref[...] = out + b2b_ref[...]


def _mlp2_call(partials, cnt2d, x, batch2d, u, w2a, b2a, w2b, b2b,
               interpret=False):
    return pl.pallas_call(
        _mlp2_body,
        grid=(N // BLK2,),
        in_specs=[
            pl.BlockSpec((NCORES, BLK2, D), lambda i: (0, i, 0)),
            pl.BlockSpec((CRB, D), lambda i: (i, 0)),
            pl.BlockSpec((BLK2, D), lambda i: (i, 0)),
            pl.BlockSpec((BLK2, 1), lambda i: (i, 0)),
            pl.BlockSpec((16, 64), lambda i: (0, 0)),
            pl.BlockSpec((2 * D + 64, 256), lambda i: (0, 0)),
            pl.BlockSpec((1, 256), lambda i: (0, 0)),
            pl.BlockSpec((256, D), lambda i: (0, 0)),
            pl.BlockSpec((1, D), lambda i: (0, 0)),
        ],
        out_specs=pl.BlockSpec((BLK2, D), lambda i: (i, 0)),
        out_shape=jax.ShapeDtypeStruct((N, D), jnp.float32),
        interpret=interpret,
    )(partials, cnt2d, x, batch2d, u, w2a, b2a, w2b, b2b)


def kernel(x, edge_index, edge_attr, u, batch,
           W1a, b1a, W1b, b1b, W2a, b2a, W2b, b2b):
    row = edge_index[0]
    col = edge_index[1]
    row3 = row.reshape(NW, NCHUNK, CHUNK)
    col3 = col.reshape(NW, NCHUNK, CHUNK)
    xg = _gather_call()(x, row3)
    h, cnt2d = _mlp1_call(xg, edge_attr, col.reshape(-1, 1), W1a,
                          b1a.reshape(1, -1), W1b, b1b.reshape(1, -1))
    zeros_init = jnp.zeros((NPAD, D), jnp.float32)
    partials = _scatter_call()(h, col3, zeros_init)
    out = _mlp2_call(partials, cnt2d, x, batch.reshape(-1, 1), u,
                     W2a, b2a.reshape(1, -1), W2b, b2b.reshape(1, -1))
    return out


# speedup vs baseline: 1.1433x; 1.1018x over previous
import functools

import jax
import jax.numpy as jnp
from jax import lax
from jax.experimental import pallas as pl
from jax.experimental.pallas import tpu as pltpu
from jax.experimental.pallas import tpu_sc as plsc

N = 10000
E = 320000
NCORES = 2
NSUB = 16
NW = NCORES * NSUB
EPW = E // NW
CHUNK = 80
NCHUNK = EPW // CHUNK
NPAD = 10240
ROWS_PER_TILE = NPAD // NSUB
D = 128
CK = 125
CR = 80


def _mesh():
    return plsc.VectorSubcoreMesh(
        core_axis_name="c", subcore_axis_name="s",
        num_cores=NCORES, num_subcores=NSUB)


NB = 2
NGRPG = NCHUNK // NB


def _gather_body(x_hbm, row3_hbm, out_hbm, xs, idx2_v, r0, r1, sem_g, sem_w):
    c = lax.axis_index("c")
    s = lax.axis_index("s")
    wid = c * NSUB + s
    base0 = wid * EPW
    rows = [r0, r1]
    pltpu.sync_copy(row3_hbm.at[wid], idx2_v)
    rbase = pl.multiple_of(s * ROWS_PER_TILE, ROWS_PER_TILE)

    def xbody(k, carry):
        rb = pl.multiple_of(rbase + k * CHUNK, CHUNK)
        pltpu.sync_copy(x_hbm.at[pl.ds(rb, CHUNK)], r0)
        pltpu.sync_copy(r0, xs.at[pl.ds(rb, CHUNK)])
        return carry

    lax.fori_loop(0, ROWS_PER_TILE // CHUNK, xbody, 0)
    plsc.subcore_barrier()

    def _g(j, b):
        pltpu.async_copy(xs.at[idx2_v.at[j]], rows[b], sem_g)

    def _wait_g(j, b):
        pltpu.make_async_copy(xs.at[idx2_v.at[j]], rows[b], sem_g).wait()

    def _wb(j, b):
        base = pl.multiple_of(base0 + j * CHUNK, CHUNK)
        pltpu.async_copy(rows[b], out_hbm.at[pl.ds(base, CHUNK)], sem_w)

    def _wait_wb(j, b):
        base = pl.multiple_of(base0 + j * CHUNK, CHUNK)
        pltpu.make_async_copy(rows[b], out_hbm.at[pl.ds(base, CHUNK)],
                              sem_w).wait()

    for b in range(NB):
        _g(b, b)

    def outer(g, carry):
        j0 = g * NB
        for b in range(NB):
            _wait_g(j0 + b, b)
            _wb(j0 + b, b)
        for b in range(NB):
            _wait_wb(j0 + b, b)
            nxt = j0 + NB + b

            @pl.when(nxt < NCHUNK)
            def _():
                _g(nxt, b)

        return carry

    lax.fori_loop(0, NGRPG, outer, 0)
    for b in range(NCHUNK - NB * NGRPG):
        j = NGRPG * NB + b
        _wait_g(j, b)
        _wb(j, b)
    for b in range(NCHUNK - NB * NGRPG):
        j = NGRPG * NB + b
        _wait_wb(j, b)


@functools.cache
def _gather_call():
    return pl.kernel(
        _gather_body,
        out_type=jax.ShapeDtypeStruct((E, D), jnp.float32),
        mesh=_mesh(),
        scratch_types=[
            pltpu.VMEM_SHARED((NPAD, D), jnp.float32),
            pltpu.VMEM((NCHUNK, CHUNK), jnp.int32),
        ] + [pltpu.VMEM((CHUNK, D), jnp.float32)] * NB + [
            pltpu.SemaphoreType.DMA,
            pltpu.SemaphoreType.DMA,
        ],
    )


NBS = 3
NGRP = 41


def _scatter_body(h_hbm, col3_hbm, zeros_hbm, out_hbm, acc, idx2_v,
                  h0, h1, h2, sem_h, sem_s):
    c = lax.axis_index("c")
    s = lax.axis_index("s")
    wid = c * NSUB + s
    rbase = pl.multiple_of(s * ROWS_PER_TILE, ROWS_PER_TILE)
    hb = [h0, h1, h2]

    def zbody(k, carry):
        rb = pl.multiple_of(rbase + k * CHUNK, CHUNK)
        pltpu.sync_copy(zeros_hbm.at[pl.ds(rb, CHUNK)], h0)
        pltpu.sync_copy(h0, acc.at[pl.ds(rb, CHUNK)])
        return carry

    lax.fori_loop(0, ROWS_PER_TILE // CHUNK, zbody, 0)
    pltpu.sync_copy(col3_hbm.at[wid], idx2_v)
    plsc.subcore_barrier()

    base0 = wid * EPW

    def _load(j, b):
        base = pl.multiple_of(base0 + j * CHUNK, CHUNK)
        pltpu.async_copy(h_hbm.at[pl.ds(base, CHUNK)], hb[b], sem_h)

    def _wait_load(j, b):
        base = pl.multiple_of(base0 + j * CHUNK, CHUNK)
        pltpu.make_async_copy(h_hbm.at[pl.ds(base, CHUNK)], hb[b],
                              sem_h).wait()

    for b in range(NBS):
        _load(b, b)

    def outer(g, carry):
        j0 = g * NBS
        for b in range(NBS):
            _wait_load(j0 + b, b)
            pltpu.async_copy(hb[b], acc.at[idx2_v.at[j0 + b]], sem_s,
                             add=True)
        for b in range(NBS):
            pltpu.make_async_copy(
                hb[b], acc.at[idx2_v.at[j0 + b]], sem_s).wait()
            nxt = j0 + NBS + b

            @pl.when(nxt < NCHUNK)
            def _():
                _load(nxt, b)

        return carry

    lax.fori_loop(0, NGRP, outer, 0)
    for b in range(NCHUNK - NBS * NGRP):
        j = NGRP * NBS + b
        _wait_load(j, b)
        pltpu.async_copy(hb[b], acc.at[idx2_v.at[j]], sem_s, add=True)
    for b in range(NCHUNK - NBS * NGRP):
        j = NGRP * NBS + b
        pltpu.make_async_copy(hb[b], acc.at[idx2_v.at[j]], sem_s).wait()
    plsc.subcore_barrier()

    def wbody(k, carry):
        rb = pl.multiple_of(rbase + k * CHUNK, CHUNK)
        pltpu.sync_copy(acc.at[pl.ds(rb, CHUNK)], h0)
        pltpu.sync_copy(h0, out_hbm.at[c, pl.ds(rb, CHUNK)])
        return carry

    lax.fori_loop(0, ROWS_PER_TILE // CHUNK, wbody, 0)


@functools.cache
def _scatter_call():
    return pl.kernel(
        _scatter_body,
        out_type=jax.ShapeDtypeStruct((NCORES, NPAD, D), jnp.float32),
        mesh=_mesh(),
        scratch_types=[
            pltpu.VMEM_SHARED((NPAD, D), jnp.float32),
            pltpu.VMEM((NCHUNK, CHUNK), jnp.int32),
        ] + [pltpu.VMEM((CHUNK, D), jnp.float32)] * NBS + [
            pltpu.SemaphoreType.DMA,
            pltpu.SemaphoreType.DMA,
        ],
    )


BLK1 = 4000


def _mlp1_body(xg_ref, ea_ref, col_ref, w1a_ref, b1a_ref, w1b_ref, b1b_ref,
               out_ref, cnt_ref):
    w1a = w1a_ref[...].astype(jnp.bfloat16)
    m = jnp.dot(xg_ref[...].astype(jnp.bfloat16), w1a[:D],
                preferred_element_type=jnp.float32)
    m = m + jnp.dot(ea_ref[...].astype(jnp.bfloat16), w1a[D:],
                    preferred_element_type=jnp.float32)
    m = jnp.maximum(m + b1a_ref[...], 0.0).astype(jnp.bfloat16)
    h = jnp.dot(m, w1b_ref[...].astype(jnp.bfloat16),
                preferred_element_type=jnp.float32)
    out_ref[...] = h + b1b_ref[...]

    col = col_ref[...]
    hi = col // CK
    lo = col - hi * CK
    oh_hi = (hi == lax.broadcasted_iota(jnp.int32, (1, CR), 1)
             ).astype(jnp.bfloat16)
    oh_lo = (lo == lax.broadcasted_iota(jnp.int32, (1, D), 1)
             ).astype(jnp.bfloat16)
    c2 = lax.dot_general(oh_hi, oh_lo, (((0,), (0,)), ((), ())),
                         preferred_element_type=jnp.float32)

    @pl.when(pl.program_id(0) == 0)
    def _init():
        cnt_ref[...] = jnp.zeros((CR, D), jnp.float32)

    cnt_ref[...] = cnt_ref[...] + c2


def _mlp1_call(xg, ea, col2d, w1a, b1a, w1b, b1b, interpret=False):
    return pl.pallas_call(
        _mlp1_body,
        grid=(E // BLK1,),
        in_specs=[
            pl.BlockSpec((BLK1, D), lambda i: (i, 0)),
            pl.BlockSpec((BLK1, 16), lambda i: (i, 0)),
            pl.BlockSpec((BLK1, 1), lambda i: (i, 0)),
            pl.BlockSpec((D + 16, 256), lambda i: (0, 0)),
            pl.BlockSpec((1, 256), lambda i: (0, 0)),
            pl.BlockSpec((256, D), lambda i: (0, 0)),
            pl.BlockSpec((1, D), lambda i: (0, 0)),
        ],
        out_specs=[
            pl.BlockSpec((BLK1, D), lambda i: (i, 0)),
            pl.BlockSpec((CR, D), lambda i: (0, 0)),
        ],
        out_shape=[
            jax.ShapeDtypeStruct((E, D), jnp.float32),
            jax.ShapeDtypeStruct((CR, D), jnp.float32),
        ],
        interpret=interpret,
    )(xg, ea, col2d, w1a, b1a, w1b, b1b)


BLK2 = 2000
CRB = BLK2 // CK


def _mlp2_body(p_ref, cnt_ref, x_ref, b_ref, u_ref, w2a_ref, b2a_ref,
               w2b_ref, b2b_ref, out_ref):
    sums = p_ref[0] + p_ref[1]
    j = lax.broadcasted_iota(jnp.int32, (BLK2, 1), 0)
    r = j // CK
    l = j - r * CK
    e1 = (r == lax.broadcasted_iota(jnp.int32, (1, CRB), 1)
          ).astype(jnp.float32)
    tmp = jnp.dot(e1, cnt_ref[...], preferred_element_type=jnp.float32)
    mask2 = l == lax.broadcasted_iota(jnp.int32, (1, D), 1)
    cnt = jnp.sum(jnp.where(mask2, tmp, 0.0), axis=1, keepdims=True)
    aggs = sums / jnp.maximum(cnt, 1.0)

    w2a = w2a_ref[...]
    m = jnp.dot(x_ref[...], w2a[:D], preferred_element_type=jnp.float32)
    m = m + jnp.dot(aggs, w2a[D:2 * D], preferred_element_type=jnp.float32)
    oh = (b_ref[...] == lax.broadcasted_iota(jnp.int32, (1, 16), 1)
          ).astype(jnp.float32)
    uw = jnp.dot(u_ref[...], w2a[2 * D:], preferred_element_type=jnp.float32)
    m = m + jnp.dot(oh, uw, preferred_element_type=jnp.float32)
    m = jnp.maximum(m + b2a_ref[...], 0.0)
    out = jnp.dot(m, w2b_ref[...], preferred_element_type=jnp.float32)
    out_ref[...] = out + b2b_ref[...]


def _mlp2_call(partials, cnt2d, x, batch2d, u, w2a, b2a, w2b, b2b,
               interpret=False):
    return pl.pallas_call(
        _mlp2_body,
        grid=(N // BLK2,),
        in_specs=[
            pl.BlockSpec((NCORES, BLK2, D), lambda i: (0, i, 0)),
            pl.BlockSpec((CRB, D), lambda i: (i, 0)),
            pl.BlockSpec((BLK2, D), lambda i: (i, 0)),
            pl.BlockSpec((BLK2, 1), lambda i: (i, 0)),
            pl.BlockSpec((16, 64), lambda i: (0, 0)),
            pl.BlockSpec((2 * D + 64, 256), lambda i: (0, 0)),
            pl.BlockSpec((1, 256), lambda i: (0, 0)),
            pl.BlockSpec((256, D), lambda i: (0, 0)),
            pl.BlockSpec((1, D), lambda i: (0, 0)),
        ],
        out_specs=pl.BlockSpec((BLK2, D), lambda i: (i, 0)),
        out_shape=jax.ShapeDtypeStruct((N, D), jnp.float32),
        interpret=interpret,
    )(partials, cnt2d, x, batch2d, u, w2a, b2a, w2b, b2b)


def kernel(x, edge_index, edge_attr, u, batch,
           W1a, b1a, W1b, b1b, W2a, b2a, W2b, b2b):
    row = edge_index[0]
    col = edge_index[1]
    row3 = row.reshape(NW, NCHUNK, CHUNK)
    col3 = col.reshape(NW, NCHUNK, CHUNK)
    xpad = jnp.concatenate(
        [x, jnp.zeros((NPAD - N, D), jnp.float32)], axis=0)
    xg = _gather_call()(xpad, row3)
    h, cnt2d = _mlp1_call(xg, edge_attr, col.reshape(-1, 1), W1a,
                          b1a.reshape(1, -1), W1b, b1b.reshape(1, -1))
    zeros_init = jnp.zeros((NPAD, D), jnp.float32)
    partials = _scatter_call()(h, col3, zeros_init)
    out = _mlp2_call(partials, cnt2d, x, batch.reshape(-1, 1), u,
                     W2a, b2a.reshape(1, -1), W2b, b2b.reshape(1, -1))
    return out


# speedup vs baseline: 1.1715x; 1.0247x over previous
import functools

import jax
import jax.numpy as jnp
from jax import lax
from jax.experimental import pallas as pl
from jax.experimental.pallas import tpu as pltpu
from jax.experimental.pallas import tpu_sc as plsc

N = 10000
E = 320000
NCORES = 2
NSUB = 16
NW = NCORES * NSUB
EPW = E // NW
CHUNK = 80
NCHUNK = EPW // CHUNK
NPAD = 10240
ROWS_PER_TILE = NPAD // NSUB
D = 128
CK = 125
CR = 80


def _mesh():
    return plsc.VectorSubcoreMesh(
        core_axis_name="c", subcore_axis_name="s",
        num_cores=NCORES, num_subcores=NSUB)


NB = 2
NGRPG = NCHUNK // NB


def _gather_body(x_hbm, row3_hbm, out_hbm, xs, idx2_v, r0, r1, sem_g, sem_w):
    c = lax.axis_index("c")
    s = lax.axis_index("s")
    wid = c * NSUB + s
    base0 = wid * EPW
    rows = [r0, r1]
    pltpu.sync_copy(row3_hbm.at[wid], idx2_v)
    rbase = pl.multiple_of(s * ROWS_PER_TILE, ROWS_PER_TILE)

    def xbody(k, carry):
        rb = pl.multiple_of(rbase + k * CHUNK, CHUNK)
        pltpu.sync_copy(x_hbm.at[pl.ds(rb, CHUNK)], r0)
        pltpu.sync_copy(r0, xs.at[pl.ds(rb, CHUNK)])
        return carry

    lax.fori_loop(0, ROWS_PER_TILE // CHUNK, xbody, 0)
    plsc.subcore_barrier()

    def _g(j, b):
        pltpu.async_copy(xs.at[idx2_v.at[j]], rows[b], sem_g)

    def _wait_g(j, b):
        pltpu.make_async_copy(xs.at[idx2_v.at[j]], rows[b], sem_g).wait()

    def _wb(j, b):
        base = pl.multiple_of(base0 + j * CHUNK, CHUNK)
        pltpu.async_copy(rows[b], out_hbm.at[pl.ds(base, CHUNK)], sem_w)

    def _wait_wb(j, b):
        base = pl.multiple_of(base0 + j * CHUNK, CHUNK)
        pltpu.make_async_copy(rows[b], out_hbm.at[pl.ds(base, CHUNK)],
                              sem_w).wait()

    for b in range(NB):
        _g(b, b)

    def outer(g, carry):
        j0 = g * NB
        for b in range(NB):
            _wait_g(j0 + b, b)
            _wb(j0 + b, b)
        for b in range(NB):
            _wait_wb(j0 + b, b)
            nxt = j0 + NB + b

            @pl.when(nxt < NCHUNK)
            def _():
                _g(nxt, b)

        return carry

    lax.fori_loop(0, NGRPG, outer, 0)
    for b in range(NCHUNK - NB * NGRPG):
        j = NGRPG * NB + b
        _wait_g(j, b)
        _wb(j, b)
    for b in range(NCHUNK - NB * NGRPG):
        j = NGRPG * NB + b
        _wait_wb(j, b)


@functools.cache
def _gather_call():
    return pl.kernel(
        _gather_body,
        out_type=jax.ShapeDtypeStruct((E, D), jnp.float32),
        mesh=_mesh(),
        scratch_types=[
            pltpu.VMEM_SHARED((NPAD, D), jnp.float32),
            pltpu.VMEM((NCHUNK, CHUNK), jnp.int32),
        ] + [pltpu.VMEM((CHUNK, D), jnp.float32)] * NB + [
            pltpu.SemaphoreType.DMA,
            pltpu.SemaphoreType.DMA,
        ],
    )


NBS = 3
NGRP = 41


def _scatter_body(h_hbm, col3_hbm, zeros_hbm, out_hbm, acc, idx2_v,
                  h0, h1, h2, sem_h, sem_s):
    c = lax.axis_index("c")
    s = lax.axis_index("s")
    wid = c * NSUB + s
    rbase = pl.multiple_of(s * ROWS_PER_TILE, ROWS_PER_TILE)
    hb = [h0, h1, h2]

    def zbody(k, carry):
        rb = pl.multiple_of(rbase + k * CHUNK, CHUNK)
        pltpu.sync_copy(zeros_hbm.at[pl.ds(rb, CHUNK)], h0)
        pltpu.sync_copy(h0, acc.at[pl.ds(rb, CHUNK)])
        return carry

    lax.fori_loop(0, ROWS_PER_TILE // CHUNK, zbody, 0)
    pltpu.sync_copy(col3_hbm.at[wid], idx2_v)
    plsc.subcore_barrier()

    base0 = wid * EPW

    def _load(j, b):
        base = pl.multiple_of(base0 + j * CHUNK, CHUNK)
        pltpu.async_copy(h_hbm.at[pl.ds(base, CHUNK)], hb[b], sem_h)

    def _wait_load(j, b):
        base = pl.multiple_of(base0 + j * CHUNK, CHUNK)
        pltpu.make_async_copy(h_hbm.at[pl.ds(base, CHUNK)], hb[b],
                              sem_h).wait()

    for b in range(NBS):
        _load(b, b)

    def outer(g, carry):
        j0 = g * NBS
        for b in range(NBS):
            _wait_load(j0 + b, b)
            pltpu.async_copy(hb[b], acc.at[idx2_v.at[j0 + b]], sem_s,
                             add=True)
        for b in range(NBS):
            pltpu.make_async_copy(
                hb[b], acc.at[idx2_v.at[j0 + b]], sem_s).wait()
            nxt = j0 + NBS + b

            @pl.when(nxt < NCHUNK)
            def _():
                _load(nxt, b)

        return carry

    lax.fori_loop(0, NGRP, outer, 0)
    for b in range(NCHUNK - NBS * NGRP):
        j = NGRP * NBS + b
        _wait_load(j, b)
        pltpu.async_copy(hb[b], acc.at[idx2_v.at[j]], sem_s, add=True)
    for b in range(NCHUNK - NBS * NGRP):
        j = NGRP * NBS + b
        pltpu.make_async_copy(hb[b], acc.at[idx2_v.at[j]], sem_s).wait()
    plsc.subcore_barrier()

    def wbody(k, carry):
        rb = pl.multiple_of(rbase + k * CHUNK, CHUNK)
        pltpu.sync_copy(acc.at[pl.ds(rb, CHUNK)], h0)
        pltpu.sync_copy(h0, out_hbm.at[c, pl.ds(rb, CHUNK)])
        return carry

    lax.fori_loop(0, ROWS_PER_TILE // CHUNK, wbody, 0)


@functools.cache
def _scatter_call():
    return pl.kernel(
        _scatter_body,
        out_type=jax.ShapeDtypeStruct((NCORES, NPAD, D), jnp.float32),
        mesh=_mesh(),
        scratch_types=[
            pltpu.VMEM_SHARED((NPAD, D), jnp.float32),
            pltpu.VMEM((NCHUNK, CHUNK), jnp.int32),
        ] + [pltpu.VMEM((CHUNK, D), jnp.float32)] * NBS + [
            pltpu.SemaphoreType.DMA,
            pltpu.SemaphoreType.DMA,
        ],
    )


BLK1 = 8000


def _mlp1_body(xg_ref, ea_ref, col_ref, w1a_ref, b1a_ref, w1b_ref, b1b_ref,
               out_ref, cnt_ref):
    w1a = w1a_ref[...].astype(jnp.bfloat16)
    m = jnp.dot(xg_ref[...].astype(jnp.bfloat16), w1a[:D],
                preferred_element_type=jnp.float32)
    m = m + jnp.dot(ea_ref[...].astype(jnp.bfloat16), w1a[D:],
                    preferred_element_type=jnp.float32)
    m = jnp.maximum(m + b1a_ref[...], 0.0).astype(jnp.bfloat16)
    h = jnp.dot(m, w1b_ref[...].astype(jnp.bfloat16),
                preferred_element_type=jnp.float32)
    out_ref[...] = h + b1b_ref[...]

    col = col_ref[...]
    hi = col // CK
    lo = col - hi * CK
    oh_hi = (hi == lax.broadcasted_iota(jnp.int32, (1, CR), 1)
             ).astype(jnp.int8)
    oh_lo = (lo == lax.broadcasted_iota(jnp.int32, (1, D), 1)
             ).astype(jnp.int8)
    c2 = lax.dot_general(oh_hi, oh_lo, (((0,), (0,)), ((), ())),
                         preferred_element_type=jnp.int32).astype(jnp.float32)

    @pl.when(pl.program_id(0) == 0)
    def _init():
        cnt_ref[...] = jnp.zeros((CR, D), jnp.float32)

    cnt_ref[...] = cnt_ref[...] + c2


def _mlp1_call(xg, ea, col2d, w1a, b1a, w1b, b1b, interpret=False):
    return pl.pallas_call(
        _mlp1_body,
        grid=(E // BLK1,),
        in_specs=[
            pl.BlockSpec((BLK1, D), lambda i: (i, 0)),
            pl.BlockSpec((BLK1, 16), lambda i: (i, 0)),
            pl.BlockSpec((BLK1, 1), lambda i: (i, 0)),
            pl.BlockSpec((D + 16, 256), lambda i: (0, 0)),
            pl.BlockSpec((1, 256), lambda i: (0, 0)),
            pl.BlockSpec((256, D), lambda i: (0, 0)),
            pl.BlockSpec((1, D), lambda i: (0, 0)),
        ],
        out_specs=[
            pl.BlockSpec((BLK1, D), lambda i: (i, 0)),
            pl.BlockSpec((CR, D), lambda i: (0, 0)),
        ],
        out_shape=[
            jax.ShapeDtypeStruct((E, D), jnp.float32),
            jax.ShapeDtypeStruct((CR, D), jnp.float32),
        ],
        interpret=interpret,
    )(xg, ea, col2d, w1a, b1a, w1b, b1b)


BLK2 = 2000
CRB = BLK2 // CK


def _mlp2_body(p_ref, cnt_ref, x_ref, b_ref, u_ref, w2a_ref, b2a_ref,
               w2b_ref, b2b_ref, out_ref):
    sums = p_ref[0] + p_ref[1]
    j = lax.broadcasted_iota(jnp.int32, (BLK2, 1), 0)
    r = j // CK
    l = j - r * CK
    e1 = (r == lax.broadcasted_iota(jnp.int32, (1, CRB), 1)
          ).astype(jnp.float32)
    tmp = jnp.dot(e1, cnt_ref[...], preferred_element_type=jnp.float32)
    mask2 = l == lax.broadcasted_iota(jnp.int32, (1, D), 1)
    cnt = jnp.sum(jnp.where(mask2, tmp, 0.0), axis=1, keepdims=True)
    aggs = sums / jnp.maximum(cnt, 1.0)

    w2a = w2a_ref[...]
    m = jnp.dot(x_ref[...], w2a[:D], preferred_element_type=jnp.float32)
    m = m + jnp.dot(aggs, w2a[D:2 * D], preferred_element_type=jnp.float32)
    oh = (b_ref[...] == lax.broadcasted_iota(jnp.int32, (1, 16), 1)
          ).astype(jnp.float32)
    uw = jnp.dot(u_ref[...], w2a[2 * D:], preferred_element_type=jnp.float32)
    m = m + jnp.dot(oh, uw, preferred_element_type=jnp.float32)
    m = jnp.maximum(m + b2a_ref[...], 0.0)
    out = jnp.dot(m, w2b_ref[...], preferred_element_type=jnp.float32)
    out_ref[...] = out + b2b_ref[...]


def _mlp2_call(partials, cnt2d, x, batch2d, u, w2a, b2a, w2b, b2b,
               interpret=False):
    return pl.pallas_call(
        _mlp2_body,
        grid=(N // BLK2,),
        in_specs=[
            pl.BlockSpec((NCORES, BLK2, D), lambda i: (0, i, 0)),
            pl.BlockSpec((CRB, D), lambda i: (i, 0)),
            pl.BlockSpec((BLK2, D), lambda i: (i, 0)),
            pl.BlockSpec((BLK2, 1), lambda i: (i, 0)),
            pl.BlockSpec((16, 64), lambda i: (0, 0)),
            pl.BlockSpec((2 * D + 64, 256), lambda i: (0, 0)),
            pl.BlockSpec((1, 256), lambda i: (0, 0)),
            pl.BlockSpec((256, D), lambda i: (0, 0)),
            pl.BlockSpec((1, D), lambda i: (0, 0)),
        ],
        out_specs=pl.BlockSpec((BLK2, D), lambda i: (i, 0)),
        out_shape=jax.ShapeDtypeStruct((N, D), jnp.float32),
        interpret=interpret,
    )(partials, cnt2d, x, batch2d, u, w2a, b2a, w2b, b2b)


def kernel(x, edge_index, edge_attr, u, batch,
           W1a, b1a, W1b, b1b, W2a, b2a, W2b, b2b):
    row = edge_index[0]
    col = edge_index[1]
    row3 = row.reshape(NW, NCHUNK, CHUNK)
    col3 = col.reshape(NW, NCHUNK, CHUNK)
    xpad = jnp.concatenate(
        [x, jnp.zeros((NPAD - N, D), jnp.float32)], axis=0)
    xg = _gather_call()(xpad, row3)
    h, cnt2d = _mlp1_call(xg, edge_attr, col.reshape(-1, 1), W1a,
                          b1a.reshape(1, -1), W1b, b1b.reshape(1, -1))
    zeros_init = jnp.zeros((NPAD, D), jnp.float32)
    partials = _scatter_call()(h, col3, zeros_init)
    out = _mlp2_call(partials, cnt2d, x, batch.reshape(-1, 1), u,
                     W2a, b2a.reshape(1, -1), W2b, b2b.reshape(1, -1))
    return out


# speedup vs baseline: 1.1930x; 1.0183x over previous
import functools

import jax
import jax.numpy as jnp
from jax import lax
from jax.experimental import pallas as pl
from jax.experimental.pallas import tpu as pltpu
from jax.experimental.pallas import tpu_sc as plsc

N = 10000
E = 320000
NCORES = 2
NSUB = 16
NW = NCORES * NSUB
EPW = E // NW
CHUNK = 80
NCHUNK = EPW // CHUNK
NPAD = 10240
ROWS_PER_TILE = NPAD // NSUB
D = 128
CK = 125
CR = 80


def _mesh():
    return plsc.VectorSubcoreMesh(
        core_axis_name="c", subcore_axis_name="s",
        num_cores=NCORES, num_subcores=NSUB)


NB = 2
NGRPG = NCHUNK // NB


def _gather_body(x_hbm, row3_hbm, out_hbm, xs, idx2_v, r0, r1, sem_g, sem_w):
    c = lax.axis_index("c")
    s = lax.axis_index("s")
    wid = c * NSUB + s
    base0 = wid * EPW
    rows = [r0, r1]
    pltpu.sync_copy(row3_hbm.at[wid], idx2_v)
    rbase = pl.multiple_of(s * ROWS_PER_TILE, ROWS_PER_TILE)

    def xbody(k, carry):
        rb = pl.multiple_of(rbase + k * CHUNK, CHUNK)
        pltpu.sync_copy(x_hbm.at[pl.ds(rb, CHUNK)], r0)
        pltpu.sync_copy(r0, xs.at[pl.ds(rb, CHUNK)])
        return carry

    lax.fori_loop(0, ROWS_PER_TILE // CHUNK, xbody, 0)
    plsc.subcore_barrier()

    def _g(j, b):
        pltpu.async_copy(xs.at[idx2_v.at[j]], rows[b], sem_g)

    def _wait_g(j, b):
        pltpu.make_async_copy(xs.at[idx2_v.at[j]], rows[b], sem_g).wait()

    def _wb(j, b):
        base = pl.multiple_of(base0 + j * CHUNK, CHUNK)
        pltpu.async_copy(rows[b], out_hbm.at[pl.ds(base, CHUNK)], sem_w)

    def _wait_wb(j, b):
        base = pl.multiple_of(base0 + j * CHUNK, CHUNK)
        pltpu.make_async_copy(rows[b], out_hbm.at[pl.ds(base, CHUNK)],
                              sem_w).wait()

    for b in range(NB):
        _g(b, b)

    def outer(g, carry):
        j0 = g * NB
        for b in range(NB):
            _wait_g(j0 + b, b)
            _wb(j0 + b, b)
        for b in range(NB):
            _wait_wb(j0 + b, b)
            nxt = j0 + NB + b

            @pl.when(nxt < NCHUNK)
            def _():
                _g(nxt, b)

        return carry

    lax.fori_loop(0, NGRPG, outer, 0)
    for b in range(NCHUNK - NB * NGRPG):
        j = NGRPG * NB + b
        _wait_g(j, b)
        _wb(j, b)
    for b in range(NCHUNK - NB * NGRPG):
        j = NGRPG * NB + b
        _wait_wb(j, b)


@functools.cache
def _gather_call():
    return pl.kernel(
        _gather_body,
        out_type=jax.ShapeDtypeStruct((E, D), jnp.float32),
        mesh=_mesh(),
        scratch_types=[
            pltpu.VMEM_SHARED((NPAD, D), jnp.float32),
            pltpu.VMEM((NCHUNK, CHUNK), jnp.int32),
        ] + [pltpu.VMEM((CHUNK, D), jnp.float32)] * NB + [
            pltpu.SemaphoreType.DMA,
            pltpu.SemaphoreType.DMA,
        ],
    )


NBS = 3
NGRP = 41


def _scatter_body(h_hbm, col3_hbm, out_hbm, acc, idx2_v,
                  h0, h1, h2, sem_h, sem_s):
    c = lax.axis_index("c")
    s = lax.axis_index("s")
    wid = c * NSUB + s
    rbase = pl.multiple_of(s * ROWS_PER_TILE, ROWS_PER_TILE)
    hb = [h0, h1, h2]

    def vzero(i, carry):
        r = i // (D // 16)
        cc = i - r * (D // 16)
        h0[r, pl.ds(cc * 16, 16)] = jnp.zeros((16,), jnp.float32)
        return carry

    lax.fori_loop(0, CHUNK * (D // 16), vzero, 0)

    def zbody(k, carry):
        rb = pl.multiple_of(rbase + k * CHUNK, CHUNK)
        pltpu.sync_copy(h0, acc.at[pl.ds(rb, CHUNK)])
        return carry

    lax.fori_loop(0, ROWS_PER_TILE // CHUNK, zbody, 0)
    pltpu.sync_copy(col3_hbm.at[wid], idx2_v)
    plsc.subcore_barrier()

    base0 = wid * EPW

    def _load(j, b):
        base = pl.multiple_of(base0 + j * CHUNK, CHUNK)
        pltpu.async_copy(h_hbm.at[pl.ds(base, CHUNK)], hb[b], sem_h)

    def _wait_load(j, b):
        base = pl.multiple_of(base0 + j * CHUNK, CHUNK)
        pltpu.make_async_copy(h_hbm.at[pl.ds(base, CHUNK)], hb[b],
                              sem_h).wait()

    for b in range(NBS):
        _load(b, b)

    def outer(g, carry):
        j0 = g * NBS
        for b in range(NBS):
            _wait_load(j0 + b, b)
            pltpu.async_copy(hb[b], acc.at[idx2_v.at[j0 + b]], sem_s,
                             add=True)
        for b in range(NBS):
            pltpu.make_async_copy(
                hb[b], acc.at[idx2_v.at[j0 + b]], sem_s).wait()
            nxt = j0 + NBS + b

            @pl.when(nxt < NCHUNK)
            def _():
                _load(nxt, b)

        return carry

    lax.fori_loop(0, NGRP, outer, 0)
    for b in range(NCHUNK - NBS * NGRP):
        j = NGRP * NBS + b
        _wait_load(j, b)
        pltpu.async_copy(hb[b], acc.at[idx2_v.at[j]], sem_s, add=True)
    for b in range(NCHUNK - NBS * NGRP):
        j = NGRP * NBS + b
        pltpu.make_async_copy(hb[b], acc.at[idx2_v.at[j]], sem_s).wait()
    plsc.subcore_barrier()

    def wbody(k, carry):
        rb = pl.multiple_of(rbase + k * CHUNK, CHUNK)
        pltpu.sync_copy(acc.at[pl.ds(rb, CHUNK)], h0)
        pltpu.sync_copy(h0, out_hbm.at[c, pl.ds(rb, CHUNK)])
        return carry

    lax.fori_loop(0, ROWS_PER_TILE // CHUNK, wbody, 0)


@functools.cache
def _scatter_call():
    return pl.kernel(
        _scatter_body,
        out_type=jax.ShapeDtypeStruct((NCORES, NPAD, D), jnp.float32),
        mesh=_mesh(),
        scratch_types=[
            pltpu.VMEM_SHARED((NPAD, D), jnp.float32),
            pltpu.VMEM((NCHUNK, CHUNK), jnp.int32),
        ] + [pltpu.VMEM((CHUNK, D), jnp.float32)] * NBS + [
            pltpu.SemaphoreType.DMA,
            pltpu.SemaphoreType.DMA,
        ],
    )


BLK1 = 8000


def _mlp1_body(xg_ref, ea_ref, col_ref, w1a_ref, b1a_ref, w1b_ref, b1b_ref,
               out_ref, cnt_ref):
    w1a = w1a_ref[...].astype(jnp.bfloat16)
    m = jnp.dot(xg_ref[...].astype(jnp.bfloat16), w1a[:D],
                preferred_element_type=jnp.float32)
    m = m + jnp.dot(ea_ref[...].astype(jnp.bfloat16), w1a[D:],
                    preferred_element_type=jnp.float32)
    m = jnp.maximum(m + b1a_ref[...], 0.0).astype(jnp.bfloat16)
    h = jnp.dot(m, w1b_ref[...].astype(jnp.bfloat16),
                preferred_element_type=jnp.float32)
    out_ref[...] = h + b1b_ref[...]

    col = col_ref[...]
    hi = col // CK
    lo = col - hi * CK
    oh_hi = (hi == lax.broadcasted_iota(jnp.int32, (1, CR), 1)
             ).astype(jnp.int8)
    oh_lo = (lo == lax.broadcasted_iota(jnp.int32, (1, D), 1)
             ).astype(jnp.int8)
    c2 = lax.dot_general(oh_hi, oh_lo, (((0,), (0,)), ((), ())),
                         preferred_element_type=jnp.int32).astype(jnp.float32)

    @pl.when(pl.program_id(0) == 0)
    def _init():
        cnt_ref[...] = jnp.zeros((CR, D), jnp.float32)

    cnt_ref[...] = cnt_ref[...] + c2


def _mlp1_call(xg, ea, col2d, w1a, b1a, w1b, b1b, interpret=False):
    return pl.pallas_call(
        _mlp1_body,
        grid=(E // BLK1,),
        in_specs=[
            pl.BlockSpec((BLK1, D), lambda i: (i, 0)),
            pl.BlockSpec((BLK1, 16), lambda i: (i, 0)),
            pl.BlockSpec((BLK1, 1), lambda i: (i, 0)),
            pl.BlockSpec((D + 16, 256), lambda i: (0, 0)),
            pl.BlockSpec((1, 256), lambda i: (0, 0)),
            pl.BlockSpec((256, D), lambda i: (0, 0)),
            pl.BlockSpec((1, D), lambda i: (0, 0)),
        ],
        out_specs=[
            pl.BlockSpec((BLK1, D), lambda i: (i, 0)),
            pl.BlockSpec((CR, D), lambda i: (0, 0)),
        ],
        out_shape=[
            jax.ShapeDtypeStruct((E, D), jnp.float32),
            jax.ShapeDtypeStruct((CR, D), jnp.float32),
        ],
        interpret=interpret,
    )(xg, ea, col2d, w1a, b1a, w1b, b1b)


BLK2 = 2000
CRB = BLK2 // CK


def _mlp2_body(p_ref, cnt_ref, x_ref, b_ref, u_ref, w2a_ref, b2a_ref,
               w2b_ref, b2b_ref, out_ref):
    sums = p_ref[0] + p_ref[1]
    j = lax.broadcasted_iota(jnp.int32, (BLK2, 1), 0)
    r = j // CK
    l = j - r * CK
    e1 = (r == lax.broadcasted_iota(jnp.int32, (1, CRB), 1)
          ).astype(jnp.float32)
    tmp = jnp.dot(e1, cnt_ref[...], preferred_element_type=jnp.float32)
    mask2 = l == lax.broadcasted_iota(jnp.int32, (1, D), 1)
    cnt = jnp.sum(jnp.where(mask2, tmp, 0.0), axis=1, keepdims=True)
    aggs = sums / jnp.maximum(cnt, 1.0)

    w2a = w2a_ref[...]
    m = jnp.dot(x_ref[...], w2a[:D], preferred_element_type=jnp.float32)
    m = m + jnp.dot(aggs, w2a[D:2 * D], preferred_element_type=jnp.float32)
    oh = (b_ref[...] == lax.broadcasted_iota(jnp.int32, (1, 16), 1)
          ).astype(jnp.float32)
    uw = jnp.dot(u_ref[...], w2a[2 * D:], preferred_element_type=jnp.float32)
    m = m + jnp.dot(oh, uw, preferred_element_type=jnp.float32)
    m = jnp.maximum(m + b2a_ref[...], 0.0)
    out = jnp.dot(m, w2b_ref[...], preferred_element_type=jnp.float32)
    out_ref[...] = out + b2b_ref[...]


def _mlp2_call(partials, cnt2d, x, batch2d, u, w2a, b2a, w2b, b2b,
               interpret=False):
    return pl.pallas_call(
        _mlp2_body,
        grid=(N // BLK2,),
        in_specs=[
            pl.BlockSpec((NCORES, BLK2, D), lambda i: (0, i, 0)),
            pl.BlockSpec((CRB, D), lambda i: (i, 0)),
            pl.BlockSpec((BLK2, D), lambda i: (i, 0)),
            pl.BlockSpec((BLK2, 1), lambda i: (i, 0)),
            pl.BlockSpec((16, 64), lambda i: (0, 0)),
            pl.BlockSpec((2 * D + 64, 256), lambda i: (0, 0)),
            pl.BlockSpec((1, 256), lambda i: (0, 0)),
            pl.BlockSpec((256, D), lambda i: (0, 0)),
            pl.BlockSpec((1, D), lambda i: (0, 0)),
        ],
        out_specs=pl.BlockSpec((BLK2, D), lambda i: (i, 0)),
        out_shape=jax.ShapeDtypeStruct((N, D), jnp.float32),
        interpret=interpret,
    )(partials, cnt2d, x, batch2d, u, w2a, b2a, w2b, b2b)


def kernel(x, edge_index, edge_attr, u, batch,
           W1a, b1a, W1b, b1b, W2a, b2a, W2b, b2b):
    row = edge_index[0]
    col = edge_index[1]
    row3 = row.reshape(NW, NCHUNK, CHUNK)
    col3 = col.reshape(NW, NCHUNK, CHUNK)
    xpad = jnp.concatenate(
        [x, jnp.zeros((NPAD - N, D), jnp.float32)], axis=0)
    xg = _gather_call()(xpad, row3)
    h, cnt2d = _mlp1_call(xg, edge_attr, col.reshape(-1, 1), W1a,
                          b1a.reshape(1, -1), W1b, b1b.reshape(1, -1))
    partials = _scatter_call()(h, col3)
    out = _mlp2_call(partials, cnt2d, x, batch.reshape(-1, 1), u,
                     W2a, b2a.reshape(1, -1), W2b, b2b.reshape(1, -1))
    return out


# speedup vs baseline: 1.1993x; 1.0053x over previous
import functools

import jax
import jax.numpy as jnp
from jax import lax
from jax.experimental import pallas as pl
from jax.experimental.pallas import tpu as pltpu
from jax.experimental.pallas import tpu_sc as plsc

N = 10000
E = 320000
NCORES = 2
NSUB = 16
NW = NCORES * NSUB
EPW = E // NW
CHUNK = 80
NCHUNK = EPW // CHUNK
NPAD = 10240
ROWS_PER_TILE = NPAD // NSUB
D = 128
CK = 125
CR = 80


def _mesh():
    return plsc.VectorSubcoreMesh(
        core_axis_name="c", subcore_axis_name="s",
        num_cores=NCORES, num_subcores=NSUB)


NB = 2
NGRPG = NCHUNK // NB


def _gather_body(x_hbm, row3_hbm, out_hbm, xs, idx2_v, r0, r1, sem_g, sem_w):
    c = lax.axis_index("c")
    s = lax.axis_index("s")
    wid = c * NSUB + s
    base0 = wid * EPW
    rows = [r0, r1]
    pltpu.sync_copy(row3_hbm.at[wid], idx2_v)
    rbase = pl.multiple_of(s * ROWS_PER_TILE, ROWS_PER_TILE)

    def xbody(k, carry):
        rb = pl.multiple_of(rbase + k * CHUNK, CHUNK)

        @pl.when(rb < N)
        def _():
            pltpu.sync_copy(x_hbm.at[pl.ds(rb, CHUNK)], r0)
            pltpu.sync_copy(r0, xs.at[pl.ds(rb, CHUNK)])

        return carry

    lax.fori_loop(0, ROWS_PER_TILE // CHUNK, xbody, 0)
    plsc.subcore_barrier()

    def _g(j, b):
        pltpu.async_copy(xs.at[idx2_v.at[j]], rows[b], sem_g)

    def _wait_g(j, b):
        pltpu.make_async_copy(xs.at[idx2_v.at[j]], rows[b], sem_g).wait()

    def _wb(j, b):
        base = pl.multiple_of(base0 + j * CHUNK, CHUNK)
        pltpu.async_copy(rows[b], out_hbm.at[pl.ds(base, CHUNK)], sem_w)

    def _wait_wb(j, b):
        base = pl.multiple_of(base0 + j * CHUNK, CHUNK)
        pltpu.make_async_copy(rows[b], out_hbm.at[pl.ds(base, CHUNK)],
                              sem_w).wait()

    for b in range(NB):
        _g(b, b)

    def outer(g, carry):
        j0 = g * NB
        for b in range(NB):
            _wait_g(j0 + b, b)
            _wb(j0 + b, b)
        for b in range(NB):
            _wait_wb(j0 + b, b)
            nxt = j0 + NB + b

            @pl.when(nxt < NCHUNK)
            def _():
                _g(nxt, b)

        return carry

    lax.fori_loop(0, NGRPG, outer, 0)
    for b in range(NCHUNK - NB * NGRPG):
        j = NGRPG * NB + b
        _wait_g(j, b)
        _wb(j, b)
    for b in range(NCHUNK - NB * NGRPG):
        j = NGRPG * NB + b
        _wait_wb(j, b)


@functools.cache
def _gather_call():
    return pl.kernel(
        _gather_body,
        out_type=jax.ShapeDtypeStruct((E, D), jnp.float32),
        mesh=_mesh(),
        scratch_types=[
            pltpu.VMEM_SHARED((NPAD, D), jnp.float32),
            pltpu.VMEM((NCHUNK, CHUNK), jnp.int32),
        ] + [pltpu.VMEM((CHUNK, D), jnp.float32)] * NB + [
            pltpu.SemaphoreType.DMA,
            pltpu.SemaphoreType.DMA,
        ],
    )


NBS = 3
NGRP = 41


def _scatter_body(h_hbm, col3_hbm, out_hbm, acc, idx2_v,
                  h0, h1, h2, sem_h, sem_s):
    c = lax.axis_index("c")
    s = lax.axis_index("s")
    wid = c * NSUB + s
    rbase = pl.multiple_of(s * ROWS_PER_TILE, ROWS_PER_TILE)
    hb = [h0, h1, h2]

    def vzero(i, carry):
        r = i // (D // 16)
        cc = i - r * (D // 16)
        h0[r, pl.ds(cc * 16, 16)] = jnp.zeros((16,), jnp.float32)
        return carry

    lax.fori_loop(0, CHUNK * (D // 16), vzero, 0)

    def zbody(k, carry):
        rb = pl.multiple_of(rbase + k * CHUNK, CHUNK)
        pltpu.sync_copy(h0, acc.at[pl.ds(rb, CHUNK)])
        return carry

    lax.fori_loop(0, ROWS_PER_TILE // CHUNK, zbody, 0)
    pltpu.sync_copy(col3_hbm.at[wid], idx2_v)
    plsc.subcore_barrier()

    base0 = wid * EPW

    def _load(j, b):
        base = pl.multiple_of(base0 + j * CHUNK, CHUNK)
        pltpu.async_copy(h_hbm.at[pl.ds(base, CHUNK)], hb[b], sem_h)

    def _wait_load(j, b):
        base = pl.multiple_of(base0 + j * CHUNK, CHUNK)
        pltpu.make_async_copy(h_hbm.at[pl.ds(base, CHUNK)], hb[b],
                              sem_h).wait()

    for b in range(NBS):
        _load(b, b)

    def outer(g, carry):
        j0 = g * NBS
        for b in range(NBS):
            _wait_load(j0 + b, b)
            pltpu.async_copy(hb[b], acc.at[idx2_v.at[j0 + b]], sem_s,
                             add=True)
        for b in range(NBS):
            pltpu.make_async_copy(
                hb[b], acc.at[idx2_v.at[j0 + b]], sem_s).wait()
            nxt = j0 + NBS + b

            @pl.when(nxt < NCHUNK)
            def _():
                _load(nxt, b)

        return carry

    lax.fori_loop(0, NGRP, outer, 0)
    for b in range(NCHUNK - NBS * NGRP):
        j = NGRP * NBS + b
        _wait_load(j, b)
        pltpu.async_copy(hb[b], acc.at[idx2_v.at[j]], sem_s, add=True)
    for b in range(NCHUNK - NBS * NGRP):
        j = NGRP * NBS + b
        pltpu.make_async_copy(hb[b], acc.at[idx2_v.at[j]], sem_s).wait()
    plsc.subcore_barrier()

    def wbody(k, carry):
        rb = pl.multiple_of(rbase + k * CHUNK, CHUNK)
        pltpu.sync_copy(acc.at[pl.ds(rb, CHUNK)], h0)
        pltpu.sync_copy(h0, out_hbm.at[c, pl.ds(rb, CHUNK)])
        return carry

    lax.fori_loop(0, ROWS_PER_TILE // CHUNK, wbody, 0)


@functools.cache
def _scatter_call():
    return pl.kernel(
        _scatter_body,
        out_type=jax.ShapeDtypeStruct((NCORES, NPAD, D), jnp.float32),
        mesh=_mesh(),
        scratch_types=[
            pltpu.VMEM_SHARED((NPAD, D), jnp.float32),
            pltpu.VMEM((NCHUNK, CHUNK), jnp.int32),
        ] + [pltpu.VMEM((CHUNK, D), jnp.float32)] * NBS + [
            pltpu.SemaphoreType.DMA,
            pltpu.SemaphoreType.DMA,
        ],
    )


BLK1 = 8000


def _mlp1_body(xg_ref, ea_ref, col_ref, w1a_ref, b1a_ref, w1b_ref, b1b_ref,
               out_ref, cnt_ref):
    w1a = w1a_ref[...].astype(jnp.bfloat16)
    m = jnp.dot(xg_ref[...].astype(jnp.bfloat16), w1a[:D],
                preferred_element_type=jnp.float32)
    m = m + jnp.dot(ea_ref[...].astype(jnp.bfloat16), w1a[D:],
                    preferred_element_type=jnp.float32)
    m = jnp.maximum(m + b1a_ref[...], 0.0).astype(jnp.bfloat16)
    h = jnp.dot(m, w1b_ref[...].astype(jnp.bfloat16),
                preferred_element_type=jnp.float32)
    out_ref[...] = h + b1b_ref[...]

    col = col_ref[...]
    hi = col // CK
    lo = col - hi * CK
    oh_hi = (hi == lax.broadcasted_iota(jnp.int32, (1, CR), 1)
             ).astype(jnp.int8)
    oh_lo = (lo == lax.broadcasted_iota(jnp.int32, (1, D), 1)
             ).astype(jnp.int8)
    c2 = lax.dot_general(oh_hi, oh_lo, (((0,), (0,)), ((), ())),
                         preferred_element_type=jnp.int32).astype(jnp.float32)

    @pl.when(pl.program_id(0) == 0)
    def _init():
        cnt_ref[...] = jnp.zeros((CR, D), jnp.float32)

    cnt_ref[...] = cnt_ref[...] + c2


def _mlp1_call(xg, ea, col2d, w1a, b1a, w1b, b1b, interpret=False):
    return pl.pallas_call(
        _mlp1_body,
        grid=(E // BLK1,),
        in_specs=[
            pl.BlockSpec((BLK1, D), lambda i: (i, 0)),
            pl.BlockSpec((BLK1, 16), lambda i: (i, 0)),
            pl.BlockSpec((BLK1, 1), lambda i: (i, 0)),
            pl.BlockSpec((D + 16, 256), lambda i: (0, 0)),
            pl.BlockSpec((1, 256), lambda i: (0, 0)),
            pl.BlockSpec((256, D), lambda i: (0, 0)),
            pl.BlockSpec((1, D), lambda i: (0, 0)),
        ],
        out_specs=[
            pl.BlockSpec((BLK1, D), lambda i: (i, 0)),
            pl.BlockSpec((CR, D), lambda i: (0, 0)),
        ],
        out_shape=[
            jax.ShapeDtypeStruct((E, D), jnp.float32),
            jax.ShapeDtypeStruct((CR, D), jnp.float32),
        ],
        interpret=interpret,
    )(xg, ea, col2d, w1a, b1a, w1b, b1b)


BLK2 = 2000
CRB = BLK2 // CK


def _mlp2_body(p_ref, cnt_ref, x_ref, b_ref, u_ref, w2a_ref, b2a_ref,
               w2b_ref, b2b_ref, out_ref):
    sums = p_ref[0] + p_ref[1]
    j = lax.broadcasted_iota(jnp.int32, (BLK2, 1), 0)
    r = j // CK
    l = j - r * CK
    e1 = (r == lax.broadcasted_iota(jnp.int32, (1, CRB), 1)
          ).astype(jnp.float32)
    tmp = jnp.dot(e1, cnt_ref[...], preferred_element_type=jnp.float32)
    mask2 = l == lax.broadcasted_iota(jnp.int32, (1, D), 1)
    cnt = jnp.sum(jnp.where(mask2, tmp, 0.0), axis=1, keepdims=True)
    aggs = sums / jnp.maximum(cnt, 1.0)

    w2a = w2a_ref[...]
    m = jnp.dot(x_ref[...], w2a[:D], preferred_element_type=jnp.float32)
    m = m + jnp.dot(aggs, w2a[D:2 * D], preferred_element_type=jnp.float32)
    oh = (b_ref[...] == lax.broadcasted_iota(jnp.int32, (1, 16), 1)
          ).astype(jnp.float32)
    uw = jnp.dot(u_ref[...], w2a[2 * D:], preferred_element_type=jnp.float32)
    m = m + jnp.dot(oh, uw, preferred_element_type=jnp.float32)
    m = jnp.maximum(m + b2a_ref[...], 0.0)
    out = jnp.dot(m, w2b_ref[...], preferred_element_type=jnp.float32)
    out_ref[...] = out + b2b_ref[...]


def _mlp2_call(partials, cnt2d, x, batch2d, u, w2a, b2a, w2b, b2b,
               interpret=False):
    return pl.pallas_call(
        _mlp2_body,
        grid=(N // BLK2,),
        in_specs=[
            pl.BlockSpec((NCORES, BLK2, D), lambda i: (0, i, 0)),
            pl.BlockSpec((CRB, D), lambda i: (i, 0)),
            pl.BlockSpec((BLK2, D), lambda i: (i, 0)),
            pl.BlockSpec((BLK2, 1), lambda i: (i, 0)),
            pl.BlockSpec((16, 64), lambda i: (0, 0)),
            pl.BlockSpec((2 * D + 64, 256), lambda i: (0, 0)),
            pl.BlockSpec((1, 256), lambda i: (0, 0)),
            pl.BlockSpec((256, D), lambda i: (0, 0)),
            pl.BlockSpec((1, D), lambda i: (0, 0)),
        ],
        out_specs=pl.BlockSpec((BLK2, D), lambda i: (i, 0)),
        out_shape=jax.ShapeDtypeStruct((N, D), jnp.float32),
        interpret=interpret,
    )(partials, cnt2d, x, batch2d, u, w2a, b2a, w2b, b2b)


def kernel(x, edge_index, edge_attr, u, batch,
           W1a, b1a, W1b, b1b, W2a, b2a, W2b, b2b):
    row = edge_index[0]
    col = edge_index[1]
    row3 = row.reshape(NW, NCHUNK, CHUNK)
    col3 = col.reshape(NW, NCHUNK, CHUNK)
    xg = _gather_call()(x, row3)
    h, cnt2d = _mlp1_call(xg, edge_attr, col.reshape(-1, 1), W1a,
                          b1a.reshape(1, -1), W1b, b1b.reshape(1, -1))
    partials = _scatter_call()(h, col3)
    out = _mlp2_call(partials, cnt2d, x, batch.reshape(-1, 1), u,
                     W2a, b2a.reshape(1, -1), W2b, b2b.reshape(1, -1))
    return out
